# baseline (speedup 1.0000x reference)
import functools

import jax
import jax.numpy as jnp
from jax import lax
from jax.experimental import pallas as pl
from jax.experimental.pallas import tpu as pltpu
from jax.experimental.pallas import tpu_sc as plsc

_B = 16384
_D = 64
_PW = 2 * _D
_NW = 32
_BPW = _B // _NW
_ICH = 128
_NCH = _BPW // _ICH
_NSUB = 2
_CPS = _NCH // _NSUB
_RPS = _BPW // _NSUB


@functools.cache
def _make_gather2():
    mesh = plsc.VectorSubcoreMesh(core_axis_name="c", subcore_axis_name="s")

    @functools.partial(
        pl.kernel,
        mesh=mesh,
        out_type=(
            jax.ShapeDtypeStruct((_B, _PW), jnp.float32),
            jax.ShapeDtypeStruct((_B, _PW), jnp.float32),
        ),
        scratch_types=[
            pltpu.VMEM((_NCH, _ICH), jnp.int32),
            pltpu.VMEM((_NCH, _ICH), jnp.int32),
            pltpu.VMEM((_RPS, _PW), jnp.float32),
            pltpu.VMEM((_RPS, _PW), jnp.float32),
            pltpu.SemaphoreType.DMA,
        ],
    )
    def gather2(utab, itab, uids, iids, u_out, i_out, uidx, iidx, urows,
                irows, sem):
        wid = lax.axis_index("s") * 2 + lax.axis_index("c")
        rbase = wid * _NCH
        pltpu.sync_copy(uids.at[pl.ds(rbase, _NCH)], uidx)
        pltpu.sync_copy(iids.at[pl.ds(rbase, _NCH)], iidx)
        for s in range(_NSUB):
            copies = []
            for j in range(_CPS):
                jj = s * _CPS + j
                copies.append(
                    pltpu.async_copy(utab.at[uidx.at[jj]],
                                     urows.at[pl.ds(j * _ICH, _ICH)], sem))
                copies.append(
                    pltpu.async_copy(itab.at[iidx.at[jj]],
                                     irows.at[pl.ds(j * _ICH, _ICH)], sem))
            for c in copies:
                c.wait()
            base = wid * _BPW + s * _RPS
            pltpu.sync_copy(urows, u_out.at[pl.ds(base, _RPS)])
            pltpu.sync_copy(irows, i_out.at[pl.ds(base, _RPS)])

    return gather2


_CHUNK = 2048


def _mlp_body(up_ref, ip_ref, upar_ref, ipar_ref, f_ref, w1_ref, b1_ref,
              w2_ref, b2_ref, o_ref):
    w1 = w1_ref[...]
    up = up_ref[...]
    ip = ip_ref[...]
    u = jnp.where(upar_ref[...] > 0.5, up[:, _D:], up[:, :_D])
    i = jnp.where(ipar_ref[...] > 0.5, ip[:, _D:], ip[:, :_D])
    h = jnp.dot(u, w1[0:_D, :], preferred_element_type=jnp.float32)
    h = h + jnp.dot(i, w1[_D:2 * _D, :], preferred_element_type=jnp.float32)
    h = h + jnp.dot(f_ref[...], w1[2 * _D:3 * _D, :],
                    preferred_element_type=jnp.float32)
    h = jnp.maximum(h + b1_ref[...], 0.0)
    z = jnp.dot(h, w2_ref[...], preferred_element_type=jnp.float32)
    z = z + b2_ref[...]
    o_ref[...] = 1.0 / (1.0 + jnp.exp(-z))


def _mlp(up, ip, upar, ipar, f, w1, b1, w2, b2):
    grid = (_B // _CHUNK,)
    return pl.pallas_call(
        _mlp_body,
        grid=grid,
        in_specs=[
            pl.BlockSpec((_CHUNK, _PW), lambda g: (g, 0)),
            pl.BlockSpec((_CHUNK, _PW), lambda g: (g, 0)),
            pl.BlockSpec((_CHUNK, 1), lambda g: (g, 0)),
            pl.BlockSpec((_CHUNK, 1), lambda g: (g, 0)),
            pl.BlockSpec((_CHUNK, _D), lambda g: (g, 0)),
            pl.BlockSpec((3 * _D, 128), lambda g: (0, 0)),
            pl.BlockSpec((1, 128), lambda g: (0, 0)),
            pl.BlockSpec((128, 1), lambda g: (0, 0)),
            pl.BlockSpec((1, 1), lambda g: (0, 0)),
        ],
        out_specs=pl.BlockSpec((_CHUNK, 1), lambda g: (g, 0)),
        out_shape=jax.ShapeDtypeStruct((_B, 1), jnp.float32),
    )(up, ip, upar, ipar, f, w1, b1, w2, b2)


def kernel(user_ids, item_ids, item_features, user_table, item_table,
           W1, b1, W2, b2):
    uid32 = user_ids.astype(jnp.int32)
    iid32 = item_ids.astype(jnp.int32)
    uids = (uid32 >> 1).reshape(_B // _ICH, _ICH)
    iids = (iid32 >> 1).reshape(_B // _ICH, _ICH)
    ut2 = user_table.reshape(user_table.shape[0] // 2, _PW)
    it2 = item_table.reshape(item_table.shape[0] // 2, _PW)
    up, ip = _make_gather2()(ut2, it2, uids, iids)
    upar = (uid32 & 1).astype(jnp.float32).reshape(_B, 1)
    ipar = (iid32 & 1).astype(jnp.float32).reshape(_B, 1)
    out = _mlp(up, ip, upar, ipar, item_features,
               W1, b1.reshape(1, 128), W2, b2.reshape(1, 1))
    return out.reshape(_B)


# speedup vs baseline: 1.2642x; 1.2642x over previous
import functools

import jax
import jax.numpy as jnp
from jax import lax
from jax.experimental import pallas as pl
from jax.experimental.pallas import tpu as pltpu
from jax.experimental.pallas import tpu_sc as plsc

_B = 16384
_D = 64
_PW = 2 * _D
_NW = 32
_BPW = _B // _NW
_ICH = 128
_NCH = _BPW // _ICH
_NSUB = 2
_CPS = _NCH // _NSUB
_RPS = _BPW // _NSUB


@functools.cache
def _make_gather2():
    mesh = plsc.VectorSubcoreMesh(core_axis_name="c", subcore_axis_name="s")

    @functools.partial(
        pl.kernel,
        mesh=mesh,
        out_type=(
            jax.ShapeDtypeStruct((_B, _PW), jnp.float32),
            jax.ShapeDtypeStruct((_B, _PW), jnp.float32),
        ),
        scratch_types=[
            pltpu.VMEM((_NCH, _ICH), jnp.int32),
            pltpu.VMEM((_NCH, _ICH), jnp.int32),
            pltpu.VMEM((_RPS, _PW), jnp.float32),
            pltpu.VMEM((_RPS, _PW), jnp.float32),
            pltpu.SemaphoreType.DMA,
        ],
    )
    def gather2(utab, itab, uids, iids, u_out, i_out, uidx, iidx, urows,
                irows, sem):
        wid = lax.axis_index("s") * 2 + lax.axis_index("c")
        rbase = wid * _NCH
        pltpu.sync_copy(uids.at[pl.ds(rbase, _NCH)], uidx)
        pltpu.sync_copy(iids.at[pl.ds(rbase, _NCH)], iidx)
        for s in range(_NSUB):
            copies = []
            for j in range(_CPS):
                jj = s * _CPS + j
                copies.append(
                    pltpu.async_copy(utab.at[uidx.at[jj]],
                                     urows.at[pl.ds(j * _ICH, _ICH)], sem))
                copies.append(
                    pltpu.async_copy(itab.at[iidx.at[jj]],
                                     irows.at[pl.ds(j * _ICH, _ICH)], sem))
            for c in copies:
                c.wait()
            base = wid * _BPW + s * _RPS
            pltpu.sync_copy(urows, u_out.at[pl.ds(base, _RPS)])
            pltpu.sync_copy(irows, i_out.at[pl.ds(base, _RPS)])

    return gather2


_TCOLS = 2048


_HB = _TCOLS // 2


def _repack_body(in_ref, out_ref):
    x = in_ref[...]
    xt = jnp.transpose(x)
    out_ref[:, 0:_D] = xt[0:_HB, :]
    out_ref[:, _D:_PW] = xt[_HB:_TCOLS, :]


def _repack(table_t):
    n = table_t.shape[1]
    steps = pl.cdiv(n, _TCOLS)
    return pl.pallas_call(
        _repack_body,
        grid=(steps,),
        in_specs=[pl.BlockSpec((_D, _TCOLS), lambda g: (0, g))],
        out_specs=pl.BlockSpec((_HB, _PW), lambda g: (g, 0)),
        out_shape=jax.ShapeDtypeStruct((steps * _HB, _PW), jnp.float32),
    )(table_t)


_CHUNK = 2048


def _mlp_body(up_ref, ip_ref, upar_ref, ipar_ref, f_ref, w1_ref, b1_ref,
              w2_ref, b2_ref, o_ref):
    w1 = w1_ref[...]
    up = up_ref[...]
    ip = ip_ref[...]
    u = jnp.where(upar_ref[...] > 0.5, up[:, _D:], up[:, :_D])
    i = jnp.where(ipar_ref[...] > 0.5, ip[:, _D:], ip[:, :_D])
    h = jnp.dot(u, w1[0:_D, :], preferred_element_type=jnp.float32)
    h = h + jnp.dot(i, w1[_D:2 * _D, :], preferred_element_type=jnp.float32)
    h = h + jnp.dot(f_ref[...], w1[2 * _D:3 * _D, :],
                    preferred_element_type=jnp.float32)
    h = jnp.maximum(h + b1_ref[...], 0.0)
    z = jnp.dot(h, w2_ref[...], preferred_element_type=jnp.float32)
    z = z + b2_ref[...]
    o_ref[...] = 1.0 / (1.0 + jnp.exp(-z))


def _mlp(up, ip, upar, ipar, f, w1, b1, w2, b2):
    grid = (_B // _CHUNK,)
    return pl.pallas_call(
        _mlp_body,
        grid=grid,
        in_specs=[
            pl.BlockSpec((_CHUNK, _PW), lambda g: (g, 0)),
            pl.BlockSpec((_CHUNK, _PW), lambda g: (g, 0)),
            pl.BlockSpec((_CHUNK, 1), lambda g: (g, 0)),
            pl.BlockSpec((_CHUNK, 1), lambda g: (g, 0)),
            pl.BlockSpec((_CHUNK, _D), lambda g: (g, 0)),
            pl.BlockSpec((3 * _D, 128), lambda g: (0, 0)),
            pl.BlockSpec((1, 128), lambda g: (0, 0)),
            pl.BlockSpec((128, 1), lambda g: (0, 0)),
            pl.BlockSpec((1, 1), lambda g: (0, 0)),
        ],
        out_specs=pl.BlockSpec((_CHUNK, 1), lambda g: (g, 0)),
        out_shape=jax.ShapeDtypeStruct((_B, 1), jnp.float32),
    )(up, ip, upar, ipar, f, w1, b1, w2, b2)


def kernel(user_ids, item_ids, item_features, user_table, item_table,
           W1, b1, W2, b2):
    uid32 = user_ids.astype(jnp.int32)
    iid32 = item_ids.astype(jnp.int32)
    upair = ((uid32 >> 11) << 10) | (uid32 & 1023)
    ipair = ((iid32 >> 11) << 10) | (iid32 & 1023)
    uids = upair.reshape(_B // _ICH, _ICH)
    iids = ipair.reshape(_B // _ICH, _ICH)
    ut2 = _repack(user_table.T)
    it2 = _repack(item_table.T)
    up, ip = _make_gather2()(ut2, it2, uids, iids)
    upar = ((uid32 >> 10) & 1).astype(jnp.float32).reshape(_B, 1)
    ipar = ((iid32 >> 10) & 1).astype(jnp.float32).reshape(_B, 1)
    out = _mlp(up, ip, upar, ipar, item_features,
               W1, b1.reshape(1, 128), W2, b2.reshape(1, 1))
    return out.reshape(_B)


# speedup vs baseline: 1.9936x; 1.5770x over previous
import functools

import jax
import jax.numpy as jnp
from jax import lax
from jax.experimental import pallas as pl
from jax.experimental.pallas import tpu as pltpu
from jax.experimental.pallas import tpu_sc as plsc

_B = 16384
_D = 64
_PW = 2 * _D
_NW = 32
_BPW = _B // _NW
_ICH = 128
_NCH = _BPW // _ICH
_NSUB = 2
_CPS = _NCH // _NSUB
_RPS = _BPW // _NSUB


@functools.cache
def _make_gather2():
    mesh = plsc.VectorSubcoreMesh(core_axis_name="c", subcore_axis_name="s")

    @functools.partial(
        pl.kernel,
        mesh=mesh,
        out_type=(
            jax.ShapeDtypeStruct((_B, _PW), jnp.float32),
            jax.ShapeDtypeStruct((_B, _PW), jnp.float32),
        ),
        scratch_types=[
            pltpu.VMEM((_NCH, _ICH), jnp.int32),
            pltpu.VMEM((_NCH, _ICH), jnp.int32),
            pltpu.VMEM((_RPS, _PW), jnp.float32),
            pltpu.VMEM((_RPS, _PW), jnp.float32),
            pltpu.SemaphoreType.DMA,
        ],
    )
    def gather2(utab, itab, uids, iids, u_out, i_out, uidx, iidx, urows,
                irows, sem):
        wid = lax.axis_index("s") * 2 + lax.axis_index("c")
        rbase = wid * _NCH
        pltpu.sync_copy(uids.at[pl.ds(rbase, _NCH)], uidx)
        pltpu.sync_copy(iids.at[pl.ds(rbase, _NCH)], iidx)
        for s in range(_NSUB):
            copies = []
            for j in range(_CPS):
                jj = s * _CPS + j
                copies.append(
                    pltpu.async_copy(utab.at[uidx.at[jj]],
                                     urows.at[pl.ds(j * _ICH, _ICH)], sem))
                copies.append(
                    pltpu.async_copy(itab.at[iidx.at[jj]],
                                     irows.at[pl.ds(j * _ICH, _ICH)], sem))
            for c in copies:
                c.wait()
            base = wid * _BPW + s * _RPS
            pltpu.sync_copy(urows, u_out.at[pl.ds(base, _RPS)])
            pltpu.sync_copy(irows, i_out.at[pl.ds(base, _RPS)])

    return gather2


_TCOLS = 8192


_HB = _TCOLS // 2
_SB = _TCOLS.bit_length() - 1


def _repack_body(in_ref, out_ref):
    x = in_ref[...]
    xt = jnp.transpose(x)
    out_ref[:, 0:_D] = xt[0:_HB, :]
    out_ref[:, _D:_PW] = xt[_HB:_TCOLS, :]


def _repack(table_t):
    n = table_t.shape[1]
    steps = pl.cdiv(n, _TCOLS)
    return pl.pallas_call(
        _repack_body,
        grid=(steps,),
        in_specs=[pl.BlockSpec((_D, _TCOLS), lambda g: (0, g))],
        out_specs=pl.BlockSpec((_HB, _PW), lambda g: (g, 0)),
        out_shape=jax.ShapeDtypeStruct((steps * _HB, _PW), jnp.float32),
    )(table_t)


_CHUNK = 2048


def _mlp_body(up_ref, ip_ref, upar_ref, ipar_ref, f_ref, w1_ref, b1_ref,
              w2_ref, b2_ref, o_ref):
    w1 = w1_ref[...]
    up = up_ref[...]
    ip = ip_ref[...]
    u = jnp.where(upar_ref[...] > 0.5, up[:, _D:], up[:, :_D])
    i = jnp.where(ipar_ref[...] > 0.5, ip[:, _D:], ip[:, :_D])
    h = jnp.dot(u, w1[0:_D, :], preferred_element_type=jnp.float32)
    h = h + jnp.dot(i, w1[_D:2 * _D, :], preferred_element_type=jnp.float32)
    h = h + jnp.dot(f_ref[...], w1[2 * _D:3 * _D, :],
                    preferred_element_type=jnp.float32)
    h = jnp.maximum(h + b1_ref[...], 0.0)
    z = jnp.dot(h, w2_ref[...], preferred_element_type=jnp.float32)
    z = z + b2_ref[...]
    o_ref[...] = 1.0 / (1.0 + jnp.exp(-z))


def _mlp(up, ip, upar, ipar, f, w1, b1, w2, b2):
    grid = (_B // _CHUNK,)
    return pl.pallas_call(
        _mlp_body,
        grid=grid,
        in_specs=[
            pl.BlockSpec((_CHUNK, _PW), lambda g: (g, 0)),
            pl.BlockSpec((_CHUNK, _PW), lambda g: (g, 0)),
            pl.BlockSpec((_CHUNK, 1), lambda g: (g, 0)),
            pl.BlockSpec((_CHUNK, 1), lambda g: (g, 0)),
            pl.BlockSpec((_CHUNK, _D), lambda g: (g, 0)),
            pl.BlockSpec((3 * _D, 128), lambda g: (0, 0)),
            pl.BlockSpec((1, 128), lambda g: (0, 0)),
            pl.BlockSpec((128, 1), lambda g: (0, 0)),
            pl.BlockSpec((1, 1), lambda g: (0, 0)),
        ],
        out_specs=pl.BlockSpec((_CHUNK, 1), lambda g: (g, 0)),
        out_shape=jax.ShapeDtypeStruct((_B, 1), jnp.float32),
    )(up, ip, upar, ipar, f, w1, b1, w2, b2)


def kernel(user_ids, item_ids, item_features, user_table, item_table,
           W1, b1, W2, b2):
    uid32 = user_ids.astype(jnp.int32)
    iid32 = item_ids.astype(jnp.int32)
    upair = ((uid32 >> _SB) << (_SB - 1)) | (uid32 & (_HB - 1))
    ipair = ((iid32 >> _SB) << (_SB - 1)) | (iid32 & (_HB - 1))
    uids = upair.reshape(_B // _ICH, _ICH)
    iids = ipair.reshape(_B // _ICH, _ICH)
    ut2 = _repack(user_table.T)
    it2 = _repack(item_table.T)
    up, ip = _make_gather2()(ut2, it2, uids, iids)
    upar = ((uid32 >> (_SB - 1)) & 1).astype(jnp.float32).reshape(_B, 1)
    ipar = ((iid32 >> (_SB - 1)) & 1).astype(jnp.float32).reshape(_B, 1)
    out = _mlp(up, ip, upar, ipar, item_features,
               W1, b1.reshape(1, 128), W2, b2.reshape(1, 1))
    return out.reshape(_B)


# speedup vs baseline: 2.2822x; 1.1448x over previous
import functools

import jax
import jax.numpy as jnp
from jax import lax
from jax.experimental import pallas as pl
from jax.experimental.pallas import tpu as pltpu
from jax.experimental.pallas import tpu_sc as plsc

_B = 16384
_D = 64
_PW = 2 * _D
_NW = 32
_BPW = _B // _NW
_ICH = 128
_NCH = _BPW // _ICH
_NSUB = 2
_CPS = _NCH // _NSUB
_RPS = _BPW // _NSUB


@functools.cache
def _make_gather2():
    mesh = plsc.VectorSubcoreMesh(core_axis_name="c", subcore_axis_name="s")

    @functools.partial(
        pl.kernel,
        mesh=mesh,
        out_type=(
            jax.ShapeDtypeStruct((_B, _PW), jnp.float32),
            jax.ShapeDtypeStruct((_B, _PW), jnp.float32),
        ),
        scratch_types=[
            pltpu.VMEM((_NCH, _ICH), jnp.int32),
            pltpu.VMEM((_NCH, _ICH), jnp.int32),
            pltpu.VMEM((_RPS, _PW), jnp.float32),
            pltpu.VMEM((_RPS, _PW), jnp.float32),
            pltpu.SemaphoreType.DMA,
        ],
    )
    def gather2(utab, itab, uids, iids, u_out, i_out, uidx, iidx, urows,
                irows, sem):
        wid = lax.axis_index("s") * 2 + lax.axis_index("c")
        rbase = wid * _NCH
        pltpu.sync_copy(uids.at[pl.ds(rbase, _NCH)], uidx)
        pltpu.sync_copy(iids.at[pl.ds(rbase, _NCH)], iidx)
        for s in range(_NSUB):
            copies = []
            for j in range(_CPS):
                jj = s * _CPS + j
                copies.append(
                    pltpu.async_copy(utab.at[uidx.at[jj]],
                                     urows.at[pl.ds(j * _ICH, _ICH)], sem))
                copies.append(
                    pltpu.async_copy(itab.at[iidx.at[jj]],
                                     irows.at[pl.ds(j * _ICH, _ICH)], sem))
            for c in copies:
                c.wait()
            base = wid * _BPW + s * _RPS
            pltpu.sync_copy(urows, u_out.at[pl.ds(base, _RPS)])
            pltpu.sync_copy(irows, i_out.at[pl.ds(base, _RPS)])

    return gather2


_TCOLS = 32768


_HB = _TCOLS // 2
_SB = _TCOLS.bit_length() - 1


def _repack_body(in_ref, out_ref):
    x = in_ref[...]
    xt = jnp.transpose(x)
    out_ref[:, 0:_D] = xt[0:_HB, :]
    out_ref[:, _D:_PW] = xt[_HB:_TCOLS, :]


def _repack(table_t):
    n = table_t.shape[1]
    steps = pl.cdiv(n, _TCOLS)
    return pl.pallas_call(
        _repack_body,
        grid=(steps,),
        in_specs=[pl.BlockSpec((_D, _TCOLS), lambda g: (0, g))],
        out_specs=pl.BlockSpec((_HB, _PW), lambda g: (g, 0)),
        out_shape=jax.ShapeDtypeStruct((steps * _HB, _PW), jnp.float32),
    )(table_t)


_CHUNK = 2048


def _mlp_body(up_ref, ip_ref, upar_ref, ipar_ref, f_ref, w1_ref, b1_ref,
              w2_ref, b2_ref, o_ref):
    w1 = w1_ref[...]
    up = up_ref[...]
    ip = ip_ref[...]
    u = jnp.where(upar_ref[...] > 0.5, up[:, _D:], up[:, :_D])
    i = jnp.where(ipar_ref[...] > 0.5, ip[:, _D:], ip[:, :_D])
    h = jnp.dot(u, w1[0:_D, :], preferred_element_type=jnp.float32)
    h = h + jnp.dot(i, w1[_D:2 * _D, :], preferred_element_type=jnp.float32)
    h = h + jnp.dot(f_ref[...], w1[2 * _D:3 * _D, :],
                    preferred_element_type=jnp.float32)
    h = jnp.maximum(h + b1_ref[...], 0.0)
    z = jnp.dot(h, w2_ref[...], preferred_element_type=jnp.float32)
    z = z + b2_ref[...]
    o_ref[...] = 1.0 / (1.0 + jnp.exp(-z))


def _mlp(up, ip, upar, ipar, f, w1, b1, w2, b2):
    grid = (_B // _CHUNK,)
    return pl.pallas_call(
        _mlp_body,
        grid=grid,
        in_specs=[
            pl.BlockSpec((_CHUNK, _PW), lambda g: (g, 0)),
            pl.BlockSpec((_CHUNK, _PW), lambda g: (g, 0)),
            pl.BlockSpec((_CHUNK, 1), lambda g: (g, 0)),
            pl.BlockSpec((_CHUNK, 1), lambda g: (g, 0)),
            pl.BlockSpec((_CHUNK, _D), lambda g: (g, 0)),
            pl.BlockSpec((3 * _D, 128), lambda g: (0, 0)),
            pl.BlockSpec((1, 128), lambda g: (0, 0)),
            pl.BlockSpec((128, 1), lambda g: (0, 0)),
            pl.BlockSpec((1, 1), lambda g: (0, 0)),
        ],
        out_specs=pl.BlockSpec((_CHUNK, 1), lambda g: (g, 0)),
        out_shape=jax.ShapeDtypeStruct((_B, 1), jnp.float32),
    )(up, ip, upar, ipar, f, w1, b1, w2, b2)


def kernel(user_ids, item_ids, item_features, user_table, item_table,
           W1, b1, W2, b2):
    uid32 = user_ids.astype(jnp.int32)
    iid32 = item_ids.astype(jnp.int32)
    upair = ((uid32 >> _SB) << (_SB - 1)) | (uid32 & (_HB - 1))
    ipair = ((iid32 >> _SB) << (_SB - 1)) | (iid32 & (_HB - 1))
    uids = upair.reshape(_B // _ICH, _ICH)
    iids = ipair.reshape(_B // _ICH, _ICH)
    ut2 = _repack(user_table.T)
    it2 = _repack(item_table.T)
    up, ip = _make_gather2()(ut2, it2, uids, iids)
    upar = ((uid32 >> (_SB - 1)) & 1).astype(jnp.float32).reshape(_B, 1)
    ipar = ((iid32 >> (_SB - 1)) & 1).astype(jnp.float32).reshape(_B, 1)
    out = _mlp(up, ip, upar, ipar, item_features,
               W1, b1.reshape(1, 128), W2, b2.reshape(1, 1))
    return out.reshape(_B)


# speedup vs baseline: 2.5205x; 1.1044x over previous
import functools

import jax
import jax.numpy as jnp
from jax import lax
from jax.experimental import pallas as pl
from jax.experimental.pallas import tpu as pltpu
from jax.experimental.pallas import tpu_sc as plsc

_B = 16384
_D = 64
_PW = 2 * _D
_NW = 32
_BPW = _B // _NW
_ICH = 128
_NCH = _BPW // _ICH
_NSUB = 2
_CPS = _NCH // _NSUB
_RPS = _BPW // _NSUB


@functools.cache
def _make_gather2():
    mesh = plsc.VectorSubcoreMesh(core_axis_name="c", subcore_axis_name="s")

    @functools.partial(
        pl.kernel,
        mesh=mesh,
        out_type=(
            jax.ShapeDtypeStruct((_B, _PW), jnp.float32),
            jax.ShapeDtypeStruct((_B, _PW), jnp.float32),
        ),
        scratch_types=[
            pltpu.VMEM((_NCH, _ICH), jnp.int32),
            pltpu.VMEM((_NCH, _ICH), jnp.int32),
            pltpu.VMEM((_RPS, _PW), jnp.float32),
            pltpu.VMEM((_RPS, _PW), jnp.float32),
            pltpu.SemaphoreType.DMA,
        ],
    )
    def gather2(utab, itab, uids, iids, u_out, i_out, uidx, iidx, urows,
                irows, sem):
        wid = lax.axis_index("s") * 2 + lax.axis_index("c")
        rbase = wid * _NCH
        pltpu.sync_copy(uids.at[pl.ds(rbase, _NCH)], uidx)
        pltpu.sync_copy(iids.at[pl.ds(rbase, _NCH)], iidx)
        for s in range(_NSUB):
            copies = []
            for j in range(_CPS):
                jj = s * _CPS + j
                copies.append(
                    pltpu.async_copy(utab.at[uidx.at[jj]],
                                     urows.at[pl.ds(j * _ICH, _ICH)], sem))
                copies.append(
                    pltpu.async_copy(itab.at[iidx.at[jj]],
                                     irows.at[pl.ds(j * _ICH, _ICH)], sem))
            for c in copies:
                c.wait()
            base = wid * _BPW + s * _RPS
            pltpu.sync_copy(urows, u_out.at[pl.ds(base, _RPS)])
            pltpu.sync_copy(irows, i_out.at[pl.ds(base, _RPS)])

    return gather2


_TCOLS = 32768


_QB = _TCOLS // 4
_SB = _TCOLS.bit_length() - 1


def _round_bf16(x):
    q = lax.bitcast_convert_type(x, jnp.uint32)
    return (q + jnp.uint32(0x7FFF) + ((q >> 16) & jnp.uint32(1))) >> 16


def _repack_body(in_ref, out_ref):
    x = in_ref[...]
    xt = jnp.transpose(x)
    a = _round_bf16(xt[0 * _QB:1 * _QB, :])
    b = _round_bf16(xt[1 * _QB:2 * _QB, :])
    c = _round_bf16(xt[2 * _QB:3 * _QB, :])
    d = _round_bf16(xt[3 * _QB:4 * _QB, :])
    out_ref[:, 0:_D] = lax.bitcast_convert_type((a << 16) | c, jnp.float32)
    out_ref[:, _D:_PW] = lax.bitcast_convert_type((b << 16) | d, jnp.float32)


def _repack(table_t):
    n = table_t.shape[1]
    steps = pl.cdiv(n, _TCOLS)
    return pl.pallas_call(
        _repack_body,
        grid=(steps,),
        in_specs=[pl.BlockSpec((_D, _TCOLS), lambda g: (0, g))],
        out_specs=pl.BlockSpec((_QB, _PW), lambda g: (g, 0)),
        out_shape=jax.ShapeDtypeStruct((steps * _QB, _PW), jnp.float32),
    )(table_t)


_CHUNK = 2048


def _unpack_quad(packed_ref, lane_ref, hi_ref):
    v = lax.bitcast_convert_type(packed_ref[...], jnp.uint32)
    sel = jnp.where(lane_ref[...] > 0.5, v[:, _D:], v[:, :_D])
    bits = jnp.where(hi_ref[...] > 0.5, sel & jnp.uint32(0xFFFF0000),
                     sel << 16)
    return lax.bitcast_convert_type(bits, jnp.float32)


def _mlp_body(up_ref, ip_ref, ulane_ref, uhi_ref, ilane_ref, ihi_ref,
              ft_ref, w1_ref, b1_ref, w2_ref, b2_ref, o_ref):
    w1 = w1_ref[...]
    u = _unpack_quad(up_ref, ulane_ref, uhi_ref)
    i = _unpack_quad(ip_ref, ilane_ref, ihi_ref)
    h = jnp.dot(u, w1[0:_D, :], preferred_element_type=jnp.float32)
    h = h + jnp.dot(i, w1[_D:2 * _D, :], preferred_element_type=jnp.float32)
    h = h + lax.dot_general(ft_ref[...], w1[2 * _D:3 * _D, :],
                            (((0,), (0,)), ((), ())),
                            preferred_element_type=jnp.float32)
    h = jnp.maximum(h + b1_ref[...], 0.0)
    z = jnp.dot(h, w2_ref[...], preferred_element_type=jnp.float32)
    z = z + b2_ref[...]
    o_ref[...] = 1.0 / (1.0 + jnp.exp(-z))


def _mlp(up, ip, ulane, uhi, ilane, ihi, f_t, w1, b1, w2, b2):
    grid = (_B // _CHUNK,)
    return pl.pallas_call(
        _mlp_body,
        grid=grid,
        in_specs=[
            pl.BlockSpec((_CHUNK, _PW), lambda g: (g, 0)),
            pl.BlockSpec((_CHUNK, _PW), lambda g: (g, 0)),
            pl.BlockSpec((_CHUNK, 1), lambda g: (g, 0)),
            pl.BlockSpec((_CHUNK, 1), lambda g: (g, 0)),
            pl.BlockSpec((_CHUNK, 1), lambda g: (g, 0)),
            pl.BlockSpec((_CHUNK, 1), lambda g: (g, 0)),
            pl.BlockSpec((_D, _CHUNK), lambda g: (0, g)),
            pl.BlockSpec((3 * _D, 128), lambda g: (0, 0)),
            pl.BlockSpec((1, 128), lambda g: (0, 0)),
            pl.BlockSpec((128, 1), lambda g: (0, 0)),
            pl.BlockSpec((1, 1), lambda g: (0, 0)),
        ],
        out_specs=pl.BlockSpec((_CHUNK, 1), lambda g: (g, 0)),
        out_shape=jax.ShapeDtypeStruct((_B, 1), jnp.float32),
    )(up, ip, ulane, uhi, ilane, ihi, f_t, w1, b1, w2, b2)


def kernel(user_ids, item_ids, item_features, user_table, item_table,
           W1, b1, W2, b2):
    uid32 = user_ids.astype(jnp.int32)
    iid32 = item_ids.astype(jnp.int32)
    uquad = ((uid32 >> _SB) << (_SB - 2)) | (uid32 & (_QB - 1))
    iquad = ((iid32 >> _SB) << (_SB - 2)) | (iid32 & (_QB - 1))
    uids = uquad.reshape(_B // _ICH, _ICH)
    iids = iquad.reshape(_B // _ICH, _ICH)
    ut2 = _repack(user_table.T)
    it2 = _repack(item_table.T)
    up, ip = _make_gather2()(ut2, it2, uids, iids)
    uslot = (uid32 >> (_SB - 2)) & 3
    islot = (iid32 >> (_SB - 2)) & 3
    ulane = (uslot & 1).astype(jnp.float32).reshape(_B, 1)
    ilane = (islot & 1).astype(jnp.float32).reshape(_B, 1)
    uhi = (1 - (uslot >> 1)).astype(jnp.float32).reshape(_B, 1)
    ihi = (1 - (islot >> 1)).astype(jnp.float32).reshape(_B, 1)
    out = _mlp(up, ip, ulane, uhi, ilane, ihi, item_features.T,
               W1, b1.reshape(1, 128), W2, b2.reshape(1, 1))
    return out.reshape(_B)


# speedup vs baseline: 2.5404x; 1.0079x over previous
import functools

import jax
import jax.numpy as jnp
from jax import lax
from jax.experimental import pallas as pl
from jax.experimental.pallas import tpu as pltpu
from jax.experimental.pallas import tpu_sc as plsc

_B = 16384
_D = 64
_PW = 2 * _D
_NW = 32
_BPW = _B // _NW
_ICH = 128
_NCH = _BPW // _ICH
_NSUB = 2
_CPS = _NCH // _NSUB
_RPS = _BPW // _NSUB


@functools.cache
def _make_gather2():
    mesh = plsc.VectorSubcoreMesh(core_axis_name="c", subcore_axis_name="s")

    @functools.partial(
        pl.kernel,
        mesh=mesh,
        out_type=(
            jax.ShapeDtypeStruct((_B, _PW), jnp.float32),
            jax.ShapeDtypeStruct((_B, _PW), jnp.float32),
        ),
        scratch_types=[
            pltpu.VMEM((_NCH, _ICH), jnp.int32),
            pltpu.VMEM((_NCH, _ICH), jnp.int32),
            pltpu.VMEM((_RPS, _PW), jnp.float32),
            pltpu.VMEM((_RPS, _PW), jnp.float32),
            pltpu.SemaphoreType.DMA,
        ],
    )
    def gather2(utab, itab, uids, iids, u_out, i_out, uidx, iidx, urows,
                irows, sem):
        wid = lax.axis_index("s") * 2 + lax.axis_index("c")
        rbase = wid * _NCH
        pltpu.sync_copy(uids.at[pl.ds(rbase, _NCH)], uidx)
        pltpu.sync_copy(iids.at[pl.ds(rbase, _NCH)], iidx)
        for s in range(_NSUB):
            copies = []
            for j in range(_CPS):
                jj = s * _CPS + j
                copies.append(
                    pltpu.async_copy(utab.at[uidx.at[jj]],
                                     urows.at[pl.ds(j * _ICH, _ICH)], sem))
                copies.append(
                    pltpu.async_copy(itab.at[iidx.at[jj]],
                                     irows.at[pl.ds(j * _ICH, _ICH)], sem))
            for c in copies:
                c.wait()
            base = wid * _BPW + s * _RPS
            pltpu.sync_copy(urows, u_out.at[pl.ds(base, _RPS)])
            pltpu.sync_copy(irows, i_out.at[pl.ds(base, _RPS)])

    return gather2


_TCOLS = 32768


_QB = _TCOLS // 4
_SB = _TCOLS.bit_length() - 1


def _rhu(x):
    q = lax.bitcast_convert_type(x, jnp.uint32)
    return q + jnp.uint32(0x8000)


def _repack_body(in_ref, out_ref):
    x = in_ref[...]
    xt = jnp.transpose(x)
    a = _rhu(xt[0 * _QB:1 * _QB, :])
    b = _rhu(xt[1 * _QB:2 * _QB, :])
    c = _rhu(xt[2 * _QB:3 * _QB, :])
    d = _rhu(xt[3 * _QB:4 * _QB, :])
    hi = jnp.uint32(0xFFFF0000)
    out_ref[:, 0:_D] = lax.bitcast_convert_type((a & hi) | (c >> 16),
                                                jnp.float32)
    out_ref[:, _D:_PW] = lax.bitcast_convert_type((b & hi) | (d >> 16),
                                                  jnp.float32)


def _repack(table_t):
    n = table_t.shape[1]
    steps = pl.cdiv(n, _TCOLS)
    return pl.pallas_call(
        _repack_body,
        grid=(steps,),
        in_specs=[pl.BlockSpec((_D, _TCOLS), lambda g: (0, g))],
        out_specs=pl.BlockSpec((_QB, _PW), lambda g: (g, 0)),
        out_shape=jax.ShapeDtypeStruct((steps * _QB, _PW), jnp.float32),
    )(table_t)


_CHUNK = 2048


def _unpack_quad(packed_ref, lane_ref, hi_ref):
    v = lax.bitcast_convert_type(packed_ref[...], jnp.uint32)
    sel = jnp.where(lane_ref[...] > 0.5, v[:, _D:], v[:, :_D])
    bits = jnp.where(hi_ref[...] > 0.5, sel & jnp.uint32(0xFFFF0000),
                     sel << 16)
    return lax.bitcast_convert_type(bits, jnp.float32)


def _mlp_body(up_ref, ip_ref, ulane_ref, uhi_ref, ilane_ref, ihi_ref,
              ft_ref, w1_ref, b1_ref, w2_ref, b2_ref, o_ref):
    w1 = w1_ref[...]
    u = _unpack_quad(up_ref, ulane_ref, uhi_ref)
    i = _unpack_quad(ip_ref, ilane_ref, ihi_ref)
    h = jnp.dot(u, w1[0:_D, :], preferred_element_type=jnp.float32)
    h = h + jnp.dot(i, w1[_D:2 * _D, :], preferred_element_type=jnp.float32)
    h = h + lax.dot_general(ft_ref[...], w1[2 * _D:3 * _D, :],
                            (((0,), (0,)), ((), ())),
                            preferred_element_type=jnp.float32)
    h = jnp.maximum(h + b1_ref[...], 0.0)
    z = jnp.dot(h, w2_ref[...], preferred_element_type=jnp.float32)
    z = z + b2_ref[...]
    o_ref[...] = 1.0 / (1.0 + jnp.exp(-z))


def _mlp(up, ip, ulane, uhi, ilane, ihi, f_t, w1, b1, w2, b2):
    grid = (_B // _CHUNK,)
    return pl.pallas_call(
        _mlp_body,
        grid=grid,
        in_specs=[
            pl.BlockSpec((_CHUNK, _PW), lambda g: (g, 0)),
            pl.BlockSpec((_CHUNK, _PW), lambda g: (g, 0)),
            pl.BlockSpec((_CHUNK, 1), lambda g: (g, 0)),
            pl.BlockSpec((_CHUNK, 1), lambda g: (g, 0)),
            pl.BlockSpec((_CHUNK, 1), lambda g: (g, 0)),
            pl.BlockSpec((_CHUNK, 1), lambda g: (g, 0)),
            pl.BlockSpec((_D, _CHUNK), lambda g: (0, g)),
            pl.BlockSpec((3 * _D, 128), lambda g: (0, 0)),
            pl.BlockSpec((1, 128), lambda g: (0, 0)),
            pl.BlockSpec((128, 1), lambda g: (0, 0)),
            pl.BlockSpec((1, 1), lambda g: (0, 0)),
        ],
        out_specs=pl.BlockSpec((_CHUNK, 1), lambda g: (g, 0)),
        out_shape=jax.ShapeDtypeStruct((_B, 1), jnp.float32),
    )(up, ip, ulane, uhi, ilane, ihi, f_t, w1, b1, w2, b2)


def kernel(user_ids, item_ids, item_features, user_table, item_table,
           W1, b1, W2, b2):
    uid32 = user_ids.astype(jnp.int32)
    iid32 = item_ids.astype(jnp.int32)
    uquad = ((uid32 >> _SB) << (_SB - 2)) | (uid32 & (_QB - 1))
    iquad = ((iid32 >> _SB) << (_SB - 2)) | (iid32 & (_QB - 1))
    uids = uquad.reshape(_B // _ICH, _ICH)
    iids = iquad.reshape(_B // _ICH, _ICH)
    ut2 = _repack(user_table.T)
    it2 = _repack(item_table.T)
    up, ip = _make_gather2()(ut2, it2, uids, iids)
    uslot = (uid32 >> (_SB - 2)) & 3
    islot = (iid32 >> (_SB - 2)) & 3
    ulane = (uslot & 1).astype(jnp.float32).reshape(_B, 1)
    ilane = (islot & 1).astype(jnp.float32).reshape(_B, 1)
    uhi = (1 - (uslot >> 1)).astype(jnp.float32).reshape(_B, 1)
    ihi = (1 - (islot >> 1)).astype(jnp.float32).reshape(_B, 1)
    out = _mlp(up, ip, ulane, uhi, ilane, ihi, item_features.T,
               W1, b1.reshape(1, 128), W2, b2.reshape(1, 1))
    return out.reshape(_B)
